# baseline (speedup 1.0000x reference)
import functools

import jax
import jax.numpy as jnp
from jax import lax
from jax.experimental import pallas as pl
from jax.experimental.pallas import tpu as pltpu
from jax.experimental.pallas import tpu_sc as plsc

N_AUTHOR = 10000
N_GROUP = 10000
D = 128
NE = 320000

NC = 2
NS = 16
NW = NC * NS
CH = 48
NCH = 212
EPT = NCH * CH
PE = NW * EPT
DP = 10240
AP = 10112
SR = AP // NS
PAD = 10000
NWT = PE // CH
BR = 1264
GB = AP // BR

_mesh = plsc.VectorSubcoreMesh(core_axis_name="c", subcore_axis_name="s")
_f32 = jnp.float32
_i32 = jnp.int32


def _zero_vmem_2d(ref, nrows):
    def body(r, carry):
        for cb in range(D // 16):
            ref[r, pl.ds(cb * 16, 16)] = jnp.zeros((16,), _f32)
        return carry
    lax.fori_loop(0, nrows, body, 0)


def _zero_vmem_1d(ref, n):
    def body(i, carry):
        ref[pl.ds(i * 16, 16)] = jnp.zeros((16,), _f32)
        return carry
    lax.fori_loop(0, n // 16, body, 0)


@functools.partial(
    pl.kernel,
    out_type=jax.ShapeDtypeStruct((NC, DP), _f32),
    mesh=_mesh,
    compiler_params=pltpu.CompilerParams(needs_layout_passes=False),
    scratch_types=[
        pltpu.VMEM((2, CH), _i32),
        pltpu.VMEM((CH,), _f32),
        pltpu.VMEM((DP // NS,), _f32),
        pltpu.VMEM_SHARED((DP,), _f32),
    ],
)
def _sc_deg(pk_hbm, deg_out, idx_v, w_v, zd, dacc):
    cid = lax.axis_index("c")
    sid = lax.axis_index("s")
    _zero_vmem_1d(zd, DP // NS)
    for g in range(CH // 16):
        w_v[pl.ds(g * 16, 16)] = jnp.full((16,), 1.0, _f32)
    pltpu.sync_copy(zd, dacc.at[pl.ds(sid * (DP // NS), DP // NS)])
    plsc.subcore_barrier()
    bc = cid * (PE // NC // CH) + sid * NCH

    def chunk(ci, carry):
        pltpu.sync_copy(pk_hbm.at[bc + ci], idx_v)
        pltpu.sync_copy(w_v, dacc.at[idx_v.at[1]], add=True)
        return carry

    lax.fori_loop(0, NCH, chunk, 0)
    plsc.subcore_barrier()
    pltpu.sync_copy(dacc.at[pl.ds(sid * (DP // NS), DP // NS)],
                    deg_out.at[cid, pl.ds(sid * (DP // NS), DP // NS)])


@functools.partial(
    pl.kernel,
    out_type=(
        jax.ShapeDtypeStruct((NC, AP, D), _f32),
        jax.ShapeDtypeStruct((NC, AP, D), _f32),
        jax.ShapeDtypeStruct((NC, AP, D), _f32),
        jax.ShapeDtypeStruct((NC, DP), _f32),
        jax.ShapeDtypeStruct((NC, DP), _f32),
    ),
    mesh=_mesh,
    compiler_params=pltpu.CompilerParams(needs_layout_passes=False),
    scratch_types=[
        pltpu.VMEM((2, CH), _i32),
        pltpu.VMEM((2, CH), _i32),
        pltpu.VMEM((1, CH), _i32),
        pltpu.VMEM((1, CH), _i32),
        pltpu.VMEM((CH, D), _f32),
        pltpu.VMEM((CH, D), _f32),
        pltpu.VMEM((CH,), _f32),
        pltpu.VMEM((CH,), _f32),
        pltpu.VMEM((DP // NS,), _f32),
        pltpu.VMEM((AP,), _f32),
        pltpu.VMEM((AP,), _f32),
        pltpu.SemaphoreType.DMA,
        pltpu.SemaphoreType.DMA,
        pltpu.SemaphoreType.DMA,
        pltpu.SemaphoreType.DMA,
        pltpu.SemaphoreType.DMA,
        pltpu.SemaphoreType.DMA,
        pltpu.SemaphoreType.DMA,
        pltpu.SemaphoreType.DMA,
        pltpu.VMEM_SHARED((AP, D), _f32),
        pltpu.VMEM_SHARED((DP,), _f32),
    ],
)
def _sc_layer(pk_aa, pk_ga, pk_ag,
              als_ga, ald_ga, als_ag, ald_ag,
              xw, hs_ga, hs_ag,
              num_gcn, num_ga, num_ag, den_ga, den_ag,
              idx0, idx1, sx0, sx1, rows0, rows1, w0, w1, zd, ta, tb,
              gi0, gi1, ii0, ii1, so0, so1, wo0, wo1, acc, dacc):
    cid = lax.axis_index("c")
    sid = lax.axis_index("s")
    _zero_vmem_1d(zd, DP // NS)
    bc = cid * (PE // NC // CH) + sid * NCH

    def run_phase(pk_hbm, tab_hbm, num_out, gat, den_out):
        _zero_vmem_2d(rows0, CH)
        for k in range(13):
            pltpu.sync_copy(rows0, acc.at[pl.ds(sid * SR + k * CH, CH)])
        pltpu.sync_copy(rows0.at[pl.ds(0, SR - 13 * CH)],
                        acc.at[pl.ds(sid * SR + 13 * CH, SR - 13 * CH)])
        if gat:
            pltpu.sync_copy(zd, dacc.at[pl.ds(sid * (DP // NS), DP // NS)])
        plsc.subcore_barrier()

        def stage_w(idx_v, w_v):
            for g in range(CH // 16):
                si16 = idx_v[0, pl.ds(g * 16, 16)]
                di16 = idx_v[1, pl.ds(g * 16, 16)]
                a = plsc.load_gather(ta, [si16])
                b = plsc.load_gather(tb, [di16])
                s = a + b
                w_v[pl.ds(g * 16, 16)] = jnp.exp(jnp.maximum(s, 0.2 * s))

        def scale_rows(rows_v, w_v):
            def rowb(r, rcarry):
                wr = plsc.load_gather(w_v, [jnp.zeros((16,), _i32) + r])
                for cb in range(D // 16):
                    sl = pl.ds(cb * 16, 16)
                    rows_v[r, sl] = rows_v[r, sl] * wr
                return rcarry
            lax.fori_loop(0, CH, rowb, 0, unroll=4)

        def halfstep(c, ixa, sxa, rowsa, wa, gia, iia, soa, woa,
                     ixb, sxb, rowsb, wb, gib, iib, sob, wob, notfirst):
            if gat:
                stage_w(ixa, wa)
            pltpu.make_async_copy(pk_hbm.at[bc], ixb, iib).wait()

            pltpu.async_copy(tab_hbm.at[ixb.at[0]], rowsb, gib)
            for g in range(CH // 16):
                sl = pl.ds(g * 16, 16)
                sxa[0, sl] = ixa[1, sl]
            pltpu.make_async_copy(tab_hbm.at[ixa.at[0]], rowsa, gia).wait()
            pltpu.async_copy(pk_hbm.at[bc + jnp.minimum(c + 2, NCH - 1)],
                             ixa, iia)
            if gat:
                scale_rows(rowsa, wa)

        def chunk(ci, carry):
            pltpu.sync_copy(pk_hbm.at[bc + ci], idx0)
            pltpu.async_copy(tab_hbm.at[idx0.at[0]], rows0, gi0)
            pltpu.make_async_copy(tab_hbm.at[idx0.at[0]], rows0, gi0).wait()
            return carry

        lax.fori_loop(0, NCH, chunk, 0)
        plsc.subcore_barrier()
        pltpu.sync_copy(acc.at[pl.ds(sid * SR, SR)],
                        num_out.at[cid, pl.ds(sid * SR, SR)])
        if gat:
            pltpu.sync_copy(dacc.at[pl.ds(sid * (DP // NS), DP // NS)],
                            den_out.at[cid, pl.ds(sid * (DP // NS), DP // NS)])
        plsc.subcore_barrier()

    run_phase(pk_aa, xw, num_gcn, False, None)
    pltpu.sync_copy(als_ga, ta)
    pltpu.sync_copy(ald_ga, tb)
    run_phase(pk_ga, hs_ga, num_ga, True, den_ga)
    pltpu.sync_copy(als_ag, ta)
    pltpu.sync_copy(ald_ag, tb)
    run_phase(pk_ag, hs_ag, num_ag, True, den_ag)


def _leaky(x):
    return jnp.maximum(x, 0.2 * x)


def _tc_pre(xa_ref, xg_ref, dega_ref, degb_ref,
            wgcn_ref, wsga_ref, wdga_ref, asga_ref, adga_ref,
            wsag_ref, wdag_ref, asag_ref, adag_ref,
            xw_o, hsga_o, hsag_o, dis_o, alsga_o, aldga_o, alsag_o, aldag_o):
    xa = xa_ref[...]
    xg = xg_ref[...]
    dis = lax.rsqrt(dega_ref[...] + degb_ref[...] + 1.0)
    dis_o[...] = dis
    xw_o[...] = dis * jnp.dot(xa, wgcn_ref[...], preferred_element_type=_f32)
    hsga = jnp.dot(xg, wsga_ref[...], preferred_element_type=_f32)
    hsga_o[...] = hsga
    hsag = jnp.dot(xa, wsag_ref[...], preferred_element_type=_f32)
    hsag_o[...] = hsag
    alsga_o[...] = jnp.sum(hsga * asga_ref[...], axis=1, keepdims=True)
    hdga = jnp.dot(xa, wdga_ref[...], preferred_element_type=_f32)
    aldga_o[...] = jnp.sum(hdga * adga_ref[...], axis=1, keepdims=True)
    alsag_o[...] = jnp.sum(hsag * asag_ref[...], axis=1, keepdims=True)
    hdag = jnp.dot(xg, wdag_ref[...], preferred_element_type=_f32)
    aldag_o[...] = jnp.sum(hdag * adag_ref[...], axis=1, keepdims=True)


def _combine(num_gcn_ref, num_ga_ref, num_ag_ref, dga_ref, dag_ref,
             xw_ref, hsga_ref, hsag_ref, dis_ref,
             alsga_ref, aldga_ref, alsag_ref, aldag_ref,
             bgcn_ref, bga_ref, bag_ref):
    dis = dis_ref[...]
    gcn = (dis * (num_gcn_ref[0] + num_gcn_ref[1] + xw_ref[...])
           + bgcn_ref[...])
    wii_ga = jnp.exp(_leaky(alsga_ref[...] + aldga_ref[...]))
    ga = ((num_ga_ref[0] + num_ga_ref[1] + wii_ga * hsga_ref[...])
          / (dga_ref[0] + dga_ref[1] + wii_ga)
          + bga_ref[...])
    wii_ag = jnp.exp(_leaky(alsag_ref[...] + aldag_ref[...]))
    ag = ((num_ag_ref[0] + num_ag_ref[1] + wii_ag * hsag_ref[...])
          / (dag_ref[0] + dag_ref[1] + wii_ag)
          + bag_ref[...])
    return gcn + ga, ag


def _tc_mid(num_gcn_ref, num_ga_ref, num_ag_ref, dga_ref, dag_ref,
            xw_ref, hsga_ref, hsag_ref, dis_ref,
            alsga_ref, aldga_ref, alsag_ref, aldag_ref,
            bgcn_ref, bga_ref, bag_ref,
            wgcn_ref, wsga_ref, wdga_ref, asga_ref, adga_ref,
            wsag_ref, wdag_ref, asag_ref, adag_ref,
            xa_o, xg_o, xw_o, hsga_o, hsag_o,
            alsga_o, aldga_o, alsag_o, aldag_o):
    xa2, xg2 = _combine(num_gcn_ref, num_ga_ref, num_ag_ref, dga_ref, dag_ref,
                        xw_ref, hsga_ref, hsag_ref, dis_ref,
                        alsga_ref, aldga_ref, alsag_ref, aldag_ref,
                        bgcn_ref, bga_ref, bag_ref)
    xa_o[...] = xa2
    xg_o[...] = xg2
    xw_o[...] = dis_ref[...] * jnp.dot(xa2, wgcn_ref[...],
                                       preferred_element_type=_f32)
    hsga = jnp.dot(xg2, wsga_ref[...], preferred_element_type=_f32)
    hsga_o[...] = hsga
    hsag = jnp.dot(xa2, wsag_ref[...], preferred_element_type=_f32)
    hsag_o[...] = hsag
    alsga_o[...] = jnp.sum(hsga * asga_ref[...], axis=1, keepdims=True)
    hdga = jnp.dot(xa2, wdga_ref[...], preferred_element_type=_f32)
    aldga_o[...] = jnp.sum(hdga * adga_ref[...], axis=1, keepdims=True)
    alsag_o[...] = jnp.sum(hsag * asag_ref[...], axis=1, keepdims=True)
    hdag = jnp.dot(xg2, wdag_ref[...], preferred_element_type=_f32)
    aldag_o[...] = jnp.sum(hdag * adag_ref[...], axis=1, keepdims=True)


def _tc_post(num_gcn_ref, num_ga_ref, num_ag_ref, dga_ref, dag_ref,
             xw_ref, hsga_ref, hsag_ref, dis_ref,
             alsga_ref, aldga_ref, alsag_ref, aldag_ref,
             bgcn_ref, bga_ref, bag_ref,
             xa_o, xg_o):
    xa2, xg2 = _combine(num_gcn_ref, num_ga_ref, num_ag_ref, dga_ref, dag_ref,
                        xw_ref, hsga_ref, hsag_ref, dis_ref,
                        alsga_ref, aldga_ref, alsag_ref, aldag_ref,
                        bgcn_ref, bga_ref, bag_ref)
    xa_o[...] = xa2
    xg_o[...] = xg2


_ND = pl.BlockSpec((BR, D), lambda i: (i, 0))
_SC1 = pl.BlockSpec((BR, 1), lambda i: (i, 0))
_NUM = pl.BlockSpec((NC, BR, D), lambda i: (0, i, 0))
_DEN = pl.BlockSpec((NC, BR, 1), lambda i: (0, i, 0))
_WB = pl.BlockSpec((D, D), lambda i: (0, 0))
_VB = pl.BlockSpec((1, D), lambda i: (0, 0))


def _sds(shape):
    return jax.ShapeDtypeStruct(shape, _f32)


def _pack_edges(e):
    ep = jnp.pad(e.astype(_i32), ((0, 0), (0, PE - NE)),
                 constant_values=PAD)
    return jnp.transpose(jnp.reshape(ep, (2, NWT, CH)), (1, 0, 2))


def _flat(x):
    return jnp.reshape(x, (AP,))


_PRE_IN = ([_ND, _ND, _SC1, _SC1] + [_WB, _WB, _WB, _VB, _VB]
           + [_WB, _WB, _VB, _VB])
_PRE_OUT = [_ND, _ND, _ND, _SC1, _SC1, _SC1, _SC1, _SC1]
_CMB_IN = ([_NUM, _NUM, _NUM, _DEN, _DEN, _ND, _ND, _ND]
           + [_SC1] * 5 + [_VB] * 3)
_MID_IN = _CMB_IN + [_WB, _WB, _WB, _VB, _VB, _WB, _WB, _VB, _VB]
_MID_OUT = [_ND, _ND, _ND, _ND, _ND, _SC1, _SC1, _SC1, _SC1]

_tc_pre_call = pl.pallas_call(
    _tc_pre, grid=(GB,), in_specs=_PRE_IN, out_specs=_PRE_OUT,
    out_shape=tuple([_sds((AP, D))] * 3 + [_sds((AP, 1))] * 5))
_tc_mid_call = pl.pallas_call(
    _tc_mid, grid=(GB,), in_specs=_MID_IN, out_specs=_MID_OUT,
    out_shape=tuple([_sds((AP, D))] * 5 + [_sds((AP, 1))] * 4))
_tc_post_call = pl.pallas_call(
    _tc_post, grid=(GB,), in_specs=_CMB_IN, out_specs=[_ND, _ND],
    out_shape=(_sds((AP, D)), _sds((AP, D))))


def kernel(x_author, x_group, edge_index_aa, edge_index_ag, edge_index_ga,
           W_gcn, b_gcn, W_src_ag, W_dst_ag, a_src_ag, a_dst_ag, b_ag,
           W_src_ga, W_dst_ga, a_src_ga, a_dst_ga, b_ga):
    pk_aa = _pack_edges(edge_index_aa)
    pk_ag = _pack_edges(edge_index_ag)
    pk_ga = _pack_edges(edge_index_ga)
    xa_p = jnp.pad(x_author, ((0, AP - N_AUTHOR), (0, 0)))
    xg_p = jnp.pad(x_group, ((0, AP - N_GROUP), (0, 0)))

    deg2 = _sc_deg(pk_aa)
    dega = jnp.reshape(deg2[0, :AP], (AP, 1))
    degb = jnp.reshape(deg2[1, :AP], (AP, 1))

    weights = (W_gcn, W_src_ga, W_dst_ga, jnp.reshape(a_src_ga, (1, D)),
               jnp.reshape(a_dst_ga, (1, D)), W_src_ag, W_dst_ag,
               jnp.reshape(a_src_ag, (1, D)), jnp.reshape(a_dst_ag, (1, D)))
    biases = (jnp.reshape(b_gcn, (1, D)), jnp.reshape(b_ga, (1, D)),
              jnp.reshape(b_ag, (1, D)))

    state = _tc_pre_call(xa_p, xg_p, dega, degb, *weights)
    for layer in range(2):
        xw, hsga, hsag, dis_c, alsga, aldga, alsag, aldag = state
        num_gcn, num_ga, num_ag, den_ga, den_ag = _sc_layer(
            pk_aa, pk_ga, pk_ag,
            _flat(alsga), _flat(aldga), _flat(alsag), _flat(aldag),
            xw, hsga, hsag)
        dga = jnp.reshape(den_ga[:, :AP], (NC, AP, 1))
        dag = jnp.reshape(den_ag[:, :AP], (NC, AP, 1))
        cargs = (num_gcn, num_ga, num_ag, dga, dag, xw, hsga, hsag,
                 dis_c, alsga, aldga, alsag, aldag) + biases
        if layer == 0:
            out = _tc_mid_call(*cargs, *weights)
            state = (out[2], out[3], out[4], dis_c,
                     out[5], out[6], out[7], out[8])
        else:
            xa_out, xg_out = _tc_post_call(*cargs)
    return (xa_out[:N_AUTHOR], xg_out[:N_GROUP])


# speedup vs baseline: 1.3254x; 1.3254x over previous
import functools

import jax
import jax.numpy as jnp
from jax import lax
from jax.experimental import pallas as pl
from jax.experimental.pallas import tpu as pltpu
from jax.experimental.pallas import tpu_sc as plsc

N_AUTHOR = 10000
N_GROUP = 10000
D = 128
NE = 320000

NC = 2
NS = 16
NW = NC * NS
CH = 96
NCH = 106
EPT = NCH * CH
PE = NW * EPT
DP = 10240
AP = 10112
SR = AP // NS
PAD = 10000
NWT = PE // CH
BR = 1264
GB = AP // BR

_mesh = plsc.VectorSubcoreMesh(core_axis_name="c", subcore_axis_name="s")
_f32 = jnp.float32
_i32 = jnp.int32


def _zero_vmem_2d(ref, nrows):
    def body(r, carry):
        for cb in range(D // 16):
            ref[r, pl.ds(cb * 16, 16)] = jnp.zeros((16,), _f32)
        return carry
    lax.fori_loop(0, nrows, body, 0)


def _zero_vmem_1d(ref, n):
    def body(i, carry):
        ref[pl.ds(i * 16, 16)] = jnp.zeros((16,), _f32)
        return carry
    lax.fori_loop(0, n // 16, body, 0)


@functools.partial(
    pl.kernel,
    out_type=jax.ShapeDtypeStruct((NC, DP), _f32),
    mesh=_mesh,
    compiler_params=pltpu.CompilerParams(needs_layout_passes=False),
    scratch_types=[
        pltpu.VMEM((2, CH), _i32),
        pltpu.VMEM((CH,), _f32),
        pltpu.VMEM((DP // NS,), _f32),
        pltpu.VMEM_SHARED((DP,), _f32),
    ],
)
def _sc_deg(pk_hbm, deg_out, idx_v, w_v, zd, dacc):
    cid = lax.axis_index("c")
    sid = lax.axis_index("s")
    _zero_vmem_1d(zd, DP // NS)
    for g in range(CH // 16):
        w_v[pl.ds(g * 16, 16)] = jnp.full((16,), 1.0, _f32)
    pltpu.sync_copy(zd, dacc.at[pl.ds(sid * (DP // NS), DP // NS)])
    plsc.subcore_barrier()
    bc = cid * (PE // NC // CH) + sid * NCH

    def chunk(ci, carry):
        pltpu.sync_copy(pk_hbm.at[bc + ci], idx_v)
        pltpu.sync_copy(w_v, dacc.at[idx_v.at[1]], add=True)
        return carry

    lax.fori_loop(0, NCH, chunk, 0)
    plsc.subcore_barrier()
    pltpu.sync_copy(dacc.at[pl.ds(sid * (DP // NS), DP // NS)],
                    deg_out.at[cid, pl.ds(sid * (DP // NS), DP // NS)])


@functools.partial(
    pl.kernel,
    out_type=(
        jax.ShapeDtypeStruct((NC, AP, D), _f32),
        jax.ShapeDtypeStruct((NC, AP, D), _f32),
        jax.ShapeDtypeStruct((NC, AP, D), _f32),
        jax.ShapeDtypeStruct((NC, DP), _f32),
        jax.ShapeDtypeStruct((NC, DP), _f32),
    ),
    mesh=_mesh,
    compiler_params=pltpu.CompilerParams(needs_layout_passes=False),
    scratch_types=[
        pltpu.VMEM((2, CH), _i32),
        pltpu.VMEM((2, CH), _i32),
        pltpu.VMEM((1, CH), _i32),
        pltpu.VMEM((1, CH), _i32),
        pltpu.VMEM((CH, D), _f32),
        pltpu.VMEM((CH, D), _f32),
        pltpu.VMEM((CH,), _f32),
        pltpu.VMEM((CH,), _f32),
        pltpu.VMEM((DP // NS,), _f32),
        pltpu.VMEM((AP,), _f32),
        pltpu.VMEM((AP,), _f32),
        pltpu.SemaphoreType.DMA,
        pltpu.SemaphoreType.DMA,
        pltpu.SemaphoreType.DMA,
        pltpu.SemaphoreType.DMA,
        pltpu.SemaphoreType.DMA,
        pltpu.SemaphoreType.DMA,
        pltpu.SemaphoreType.DMA,
        pltpu.SemaphoreType.DMA,
        pltpu.VMEM_SHARED((AP, D), _f32),
        pltpu.VMEM_SHARED((DP,), _f32),
    ],
)
def _sc_layer(pk_aa, pk_ga, pk_ag,
              als_ga, ald_ga, als_ag, ald_ag,
              xw, hs_ga, hs_ag,
              num_gcn, num_ga, num_ag, den_ga, den_ag,
              idx0, idx1, sx0, sx1, rows0, rows1, w0, w1, zd, ta, tb,
              gi0, gi1, ii0, ii1, so0, so1, wo0, wo1, acc, dacc):
    cid = lax.axis_index("c")
    sid = lax.axis_index("s")
    _zero_vmem_1d(zd, DP // NS)
    bc = cid * (PE // NC // CH) + sid * NCH

    def run_phase(pk_hbm, tab_hbm, num_out, gat, den_out):
        _zero_vmem_2d(rows0, CH)
        for k in range(6):
            pltpu.sync_copy(rows0, acc.at[pl.ds(sid * SR + k * CH, CH)])
        pltpu.sync_copy(rows0.at[pl.ds(0, SR - 6 * CH)],
                        acc.at[pl.ds(sid * SR + 6 * CH, SR - 6 * CH)])
        if gat:
            pltpu.sync_copy(zd, dacc.at[pl.ds(sid * (DP // NS), DP // NS)])
        plsc.subcore_barrier()

        def stage_w(idx_v, w_v):
            for g in range(CH // 16):
                si16 = idx_v[0, pl.ds(g * 16, 16)]
                di16 = idx_v[1, pl.ds(g * 16, 16)]
                a = plsc.load_gather(ta, [si16])
                b = plsc.load_gather(tb, [di16])
                s = a + b
                w_v[pl.ds(g * 16, 16)] = jnp.exp(jnp.maximum(s, 0.2 * s))

        def scale_rows(rows_v, w_v):
            def rowb(r, rcarry):
                wr = plsc.load_gather(w_v, [jnp.zeros((16,), _i32) + r])
                for cb in range(D // 16):
                    sl = pl.ds(cb * 16, 16)
                    rows_v[r, sl] = rows_v[r, sl] * wr
                return rcarry
            lax.fori_loop(0, CH, rowb, 0, unroll=4)

        def halfstep(c, ixa, sxa, rowsa, wa, gia, iia, soa, woa,
                     ixb, sxb, rowsb, wb, gib, iib, sob, wob, notfirst):
            if gat:
                stage_w(ixa, wa)
            pltpu.make_async_copy(pk_hbm.at[bc], ixb, iib).wait()

            def drain_b():
                pltpu.make_async_copy(rowsb, acc.at[sxb.at[0]], sob).wait()
                if gat:
                    pltpu.make_async_copy(wb, dacc.at[sxb.at[0]], wob).wait()
            if notfirst is None:
                drain_b()
            else:
                pl.when(notfirst)(drain_b)
            pltpu.async_copy(tab_hbm.at[ixb.at[0]], rowsb, gib)
            for g in range(CH // 16):
                sl = pl.ds(g * 16, 16)
                sxa[0, sl] = ixa[1, sl]
            pltpu.make_async_copy(tab_hbm.at[ixa.at[0]], rowsa, gia).wait()
            pltpu.async_copy(pk_hbm.at[bc + jnp.minimum(c + 2, NCH - 1)],
                             ixa, iia)
            if gat:
                scale_rows(rowsa, wa)
            pltpu.async_copy(rowsa, acc.at[sxa.at[0]], soa, add=True)
            if gat:
                pltpu.async_copy(wa, dacc.at[sxa.at[0]], woa, add=True)

        pltpu.sync_copy(pk_hbm.at[bc], idx0)
        pltpu.async_copy(tab_hbm.at[idx0.at[0]], rows0, gi0)
        pltpu.async_copy(pk_hbm.at[bc + 1], idx1, ii1)

        def chunk2(j, carry):
            c0 = 2 * j
            halfstep(c0, idx0, sx0, rows0, w0, gi0, ii0, so0, wo0,
                     idx1, sx1, rows1, w1, gi1, ii1, so1, wo1, j > 0)
            halfstep(c0 + 1, idx1, sx1, rows1, w1, gi1, ii1, so1, wo1,
                     idx0, sx0, rows0, w0, gi0, ii0, so0, wo0, None)
            return carry

        lax.fori_loop(0, NCH // 2, chunk2, 0)
        pltpu.make_async_copy(pk_hbm.at[bc], idx1, ii1).wait()
        pltpu.make_async_copy(tab_hbm.at[idx0.at[0]], rows0, gi0).wait()
        pltpu.make_async_copy(rows1, acc.at[sx1.at[0]], so1).wait()
        if gat:
            pltpu.make_async_copy(w1, dacc.at[sx1.at[0]], wo1).wait()
        plsc.subcore_barrier()
        pltpu.sync_copy(acc.at[pl.ds(sid * SR, SR)],
                        num_out.at[cid, pl.ds(sid * SR, SR)])
        if gat:
            pltpu.sync_copy(dacc.at[pl.ds(sid * (DP // NS), DP // NS)],
                            den_out.at[cid, pl.ds(sid * (DP // NS), DP // NS)])
        plsc.subcore_barrier()

    run_phase(pk_aa, xw, num_gcn, False, None)
    pltpu.sync_copy(als_ga, ta)
    pltpu.sync_copy(ald_ga, tb)
    run_phase(pk_ga, hs_ga, num_ga, True, den_ga)
    pltpu.sync_copy(als_ag, ta)
    pltpu.sync_copy(ald_ag, tb)
    run_phase(pk_ag, hs_ag, num_ag, True, den_ag)


def _leaky(x):
    return jnp.maximum(x, 0.2 * x)


def _tc_pre(xa_ref, xg_ref, dega_ref, degb_ref,
            wgcn_ref, wsga_ref, wdga_ref, asga_ref, adga_ref,
            wsag_ref, wdag_ref, asag_ref, adag_ref,
            xw_o, hsga_o, hsag_o, dis_o, alsga_o, aldga_o, alsag_o, aldag_o):
    xa = xa_ref[...]
    xg = xg_ref[...]
    dis = lax.rsqrt(dega_ref[...] + degb_ref[...] + 1.0)
    dis_o[...] = dis
    xw_o[...] = dis * jnp.dot(xa, wgcn_ref[...], preferred_element_type=_f32)
    hsga = jnp.dot(xg, wsga_ref[...], preferred_element_type=_f32)
    hsga_o[...] = hsga
    hsag = jnp.dot(xa, wsag_ref[...], preferred_element_type=_f32)
    hsag_o[...] = hsag
    alsga_o[...] = jnp.sum(hsga * asga_ref[...], axis=1, keepdims=True)
    hdga = jnp.dot(xa, wdga_ref[...], preferred_element_type=_f32)
    aldga_o[...] = jnp.sum(hdga * adga_ref[...], axis=1, keepdims=True)
    alsag_o[...] = jnp.sum(hsag * asag_ref[...], axis=1, keepdims=True)
    hdag = jnp.dot(xg, wdag_ref[...], preferred_element_type=_f32)
    aldag_o[...] = jnp.sum(hdag * adag_ref[...], axis=1, keepdims=True)


def _combine(num_gcn_ref, num_ga_ref, num_ag_ref, dga_ref, dag_ref,
             xw_ref, hsga_ref, hsag_ref, dis_ref,
             alsga_ref, aldga_ref, alsag_ref, aldag_ref,
             bgcn_ref, bga_ref, bag_ref):
    dis = dis_ref[...]
    gcn = (dis * (num_gcn_ref[0] + num_gcn_ref[1] + xw_ref[...])
           + bgcn_ref[...])
    wii_ga = jnp.exp(_leaky(alsga_ref[...] + aldga_ref[...]))
    ga = ((num_ga_ref[0] + num_ga_ref[1] + wii_ga * hsga_ref[...])
          / (dga_ref[0] + dga_ref[1] + wii_ga)
          + bga_ref[...])
    wii_ag = jnp.exp(_leaky(alsag_ref[...] + aldag_ref[...]))
    ag = ((num_ag_ref[0] + num_ag_ref[1] + wii_ag * hsag_ref[...])
          / (dag_ref[0] + dag_ref[1] + wii_ag)
          + bag_ref[...])
    return gcn + ga, ag


def _tc_mid(num_gcn_ref, num_ga_ref, num_ag_ref, dga_ref, dag_ref,
            xw_ref, hsga_ref, hsag_ref, dis_ref,
            alsga_ref, aldga_ref, alsag_ref, aldag_ref,
            bgcn_ref, bga_ref, bag_ref,
            wgcn_ref, wsga_ref, wdga_ref, asga_ref, adga_ref,
            wsag_ref, wdag_ref, asag_ref, adag_ref,
            xa_o, xg_o, xw_o, hsga_o, hsag_o,
            alsga_o, aldga_o, alsag_o, aldag_o):
    xa2, xg2 = _combine(num_gcn_ref, num_ga_ref, num_ag_ref, dga_ref, dag_ref,
                        xw_ref, hsga_ref, hsag_ref, dis_ref,
                        alsga_ref, aldga_ref, alsag_ref, aldag_ref,
                        bgcn_ref, bga_ref, bag_ref)
    xa_o[...] = xa2
    xg_o[...] = xg2
    xw_o[...] = dis_ref[...] * jnp.dot(xa2, wgcn_ref[...],
                                       preferred_element_type=_f32)
    hsga = jnp.dot(xg2, wsga_ref[...], preferred_element_type=_f32)
    hsga_o[...] = hsga
    hsag = jnp.dot(xa2, wsag_ref[...], preferred_element_type=_f32)
    hsag_o[...] = hsag
    alsga_o[...] = jnp.sum(hsga * asga_ref[...], axis=1, keepdims=True)
    hdga = jnp.dot(xa2, wdga_ref[...], preferred_element_type=_f32)
    aldga_o[...] = jnp.sum(hdga * adga_ref[...], axis=1, keepdims=True)
    alsag_o[...] = jnp.sum(hsag * asag_ref[...], axis=1, keepdims=True)
    hdag = jnp.dot(xg2, wdag_ref[...], preferred_element_type=_f32)
    aldag_o[...] = jnp.sum(hdag * adag_ref[...], axis=1, keepdims=True)


def _tc_post(num_gcn_ref, num_ga_ref, num_ag_ref, dga_ref, dag_ref,
             xw_ref, hsga_ref, hsag_ref, dis_ref,
             alsga_ref, aldga_ref, alsag_ref, aldag_ref,
             bgcn_ref, bga_ref, bag_ref,
             xa_o, xg_o):
    xa2, xg2 = _combine(num_gcn_ref, num_ga_ref, num_ag_ref, dga_ref, dag_ref,
                        xw_ref, hsga_ref, hsag_ref, dis_ref,
                        alsga_ref, aldga_ref, alsag_ref, aldag_ref,
                        bgcn_ref, bga_ref, bag_ref)
    xa_o[...] = xa2
    xg_o[...] = xg2


_ND = pl.BlockSpec((BR, D), lambda i: (i, 0))
_SC1 = pl.BlockSpec((BR, 1), lambda i: (i, 0))
_NUM = pl.BlockSpec((NC, BR, D), lambda i: (0, i, 0))
_DEN = pl.BlockSpec((NC, BR, 1), lambda i: (0, i, 0))
_WB = pl.BlockSpec((D, D), lambda i: (0, 0))
_VB = pl.BlockSpec((1, D), lambda i: (0, 0))


def _sds(shape):
    return jax.ShapeDtypeStruct(shape, _f32)


def _pack_edges(e):
    ep = jnp.pad(e.astype(_i32), ((0, 0), (0, PE - NE)),
                 constant_values=PAD)
    return jnp.transpose(jnp.reshape(ep, (2, NWT, CH)), (1, 0, 2))


def _flat(x):
    return jnp.reshape(x, (AP,))


_PRE_IN = ([_ND, _ND, _SC1, _SC1] + [_WB, _WB, _WB, _VB, _VB]
           + [_WB, _WB, _VB, _VB])
_PRE_OUT = [_ND, _ND, _ND, _SC1, _SC1, _SC1, _SC1, _SC1]
_CMB_IN = ([_NUM, _NUM, _NUM, _DEN, _DEN, _ND, _ND, _ND]
           + [_SC1] * 5 + [_VB] * 3)
_MID_IN = _CMB_IN + [_WB, _WB, _WB, _VB, _VB, _WB, _WB, _VB, _VB]
_MID_OUT = [_ND, _ND, _ND, _ND, _ND, _SC1, _SC1, _SC1, _SC1]

_tc_pre_call = pl.pallas_call(
    _tc_pre, grid=(GB,), in_specs=_PRE_IN, out_specs=_PRE_OUT,
    out_shape=tuple([_sds((AP, D))] * 3 + [_sds((AP, 1))] * 5))
_tc_mid_call = pl.pallas_call(
    _tc_mid, grid=(GB,), in_specs=_MID_IN, out_specs=_MID_OUT,
    out_shape=tuple([_sds((AP, D))] * 5 + [_sds((AP, 1))] * 4))
_tc_post_call = pl.pallas_call(
    _tc_post, grid=(GB,), in_specs=_CMB_IN, out_specs=[_ND, _ND],
    out_shape=(_sds((AP, D)), _sds((AP, D))))


def kernel(x_author, x_group, edge_index_aa, edge_index_ag, edge_index_ga,
           W_gcn, b_gcn, W_src_ag, W_dst_ag, a_src_ag, a_dst_ag, b_ag,
           W_src_ga, W_dst_ga, a_src_ga, a_dst_ga, b_ga):
    pk_aa = _pack_edges(edge_index_aa)
    pk_ag = _pack_edges(edge_index_ag)
    pk_ga = _pack_edges(edge_index_ga)
    xa_p = jnp.pad(x_author, ((0, AP - N_AUTHOR), (0, 0)))
    xg_p = jnp.pad(x_group, ((0, AP - N_GROUP), (0, 0)))

    deg2 = _sc_deg(pk_aa)
    dega = jnp.reshape(deg2[0, :AP], (AP, 1))
    degb = jnp.reshape(deg2[1, :AP], (AP, 1))

    weights = (W_gcn, W_src_ga, W_dst_ga, jnp.reshape(a_src_ga, (1, D)),
               jnp.reshape(a_dst_ga, (1, D)), W_src_ag, W_dst_ag,
               jnp.reshape(a_src_ag, (1, D)), jnp.reshape(a_dst_ag, (1, D)))
    biases = (jnp.reshape(b_gcn, (1, D)), jnp.reshape(b_ga, (1, D)),
              jnp.reshape(b_ag, (1, D)))

    state = _tc_pre_call(xa_p, xg_p, dega, degb, *weights)
    for layer in range(2):
        xw, hsga, hsag, dis_c, alsga, aldga, alsag, aldag = state
        num_gcn, num_ga, num_ag, den_ga, den_ag = _sc_layer(
            pk_aa, pk_ga, pk_ag,
            _flat(alsga), _flat(aldga), _flat(alsag), _flat(aldag),
            xw, hsga, hsag)
        dga = jnp.reshape(den_ga[:, :AP], (NC, AP, 1))
        dag = jnp.reshape(den_ag[:, :AP], (NC, AP, 1))
        cargs = (num_gcn, num_ga, num_ag, dga, dag, xw, hsga, hsag,
                 dis_c, alsga, aldga, alsag, aldag) + biases
        if layer == 0:
            out = _tc_mid_call(*cargs, *weights)
            state = (out[2], out[3], out[4], dis_c,
                     out[5], out[6], out[7], out[8])
        else:
            xa_out, xg_out = _tc_post_call(*cargs)
    return (xa_out[:N_AUTHOR], xg_out[:N_GROUP])
